# baseline (speedup 1.0000x reference)
import functools
import math

import jax
import jax.numpy as jnp
from jax import lax
from jax.experimental import pallas as pl
from jax.experimental.pallas import tpu as pltpu
from jax.experimental.pallas import tpu_sc as plsc

_VOCAB = 1000000
_EMBED_DIM = 64
_SCALE = math.sqrt(_EMBED_DIM)

_CHUNK = 128
_NBUF = 2


def _embed_body(table_hbm, idx_hbm, out_hbm,
                idx_v, rows_v, outs_v, gsems, wsems,
                *, b_per_w, nc):
  wid = lax.axis_index("s") * nc + lax.axis_index("c")
  base = wid * b_per_w
  n_chunks = b_per_w // _CHUNK

  pltpu.sync_copy(idx_hbm.at[pl.ds(base, b_per_w)], idx_v)

  def idx_slice(c):
    return idx_v.at[pl.ds(c * _CHUNK, _CHUNK)]

  for b in range(_NBUF):
    pltpu.async_copy(table_hbm.at[idx_slice(b)], rows_v[b], gsems[b])

  @pl.loop(0, n_chunks // _NBUF)
  def _pipeline(i):
    for b in range(_NBUF):
      c = i * _NBUF + b
      pltpu.make_async_copy(table_hbm.at[idx_slice(c)], rows_v[b],
                            gsems[b]).wait()

      @pl.when(c >= _NBUF)
      def _():
        pltpu.make_async_copy(
            outs_v[b], out_hbm.at[pl.ds(base, _CHUNK)], wsems[b]).wait()

      @pl.loop(0, _CHUNK)
      def _scale(r):
        for k in range(_EMBED_DIM // 16):
          outs_v[b][r, pl.ds(k * 16, 16)] = (
              rows_v[b][r, pl.ds(k * 16, 16)] * _SCALE)

      @pl.when(c + _NBUF < n_chunks)
      def _():
        pltpu.async_copy(table_hbm.at[idx_slice(c + _NBUF)], rows_v[b],
                         gsems[b])

      pltpu.async_copy(
          outs_v[b], out_hbm.at[pl.ds(base + c * _CHUNK, _CHUNK)], wsems[b])

  for b in range(_NBUF):
    pltpu.make_async_copy(
        outs_v[b], out_hbm.at[pl.ds(base, _CHUNK)], wsems[b]).wait()


@jax.jit
def _embed(x_flat, table):
  info = plsc.get_sparse_core_info()
  nc, ns = info.num_cores, info.num_subcores
  nw = nc * ns
  n = x_flat.shape[0]
  assert n % (nw * _CHUNK) == 0
  b_per_w = n // nw

  mesh = plsc.VectorSubcoreMesh(
      core_axis_name="c", subcore_axis_name="s",
      num_cores=nc, num_subcores=ns)

  kern = pl.kernel(
      functools.partial(_embed_body, b_per_w=b_per_w, nc=nc),
      out_type=jax.ShapeDtypeStruct((n, _EMBED_DIM), jnp.float32),
      mesh=mesh,
      scratch_types=dict(
          idx_v=pltpu.VMEM((b_per_w,), jnp.int32),
          rows_v=[pltpu.VMEM((_CHUNK, _EMBED_DIM), jnp.float32)
                  for _ in range(_NBUF)],
          outs_v=[pltpu.VMEM((_CHUNK, _EMBED_DIM), jnp.float32)
                  for _ in range(_NBUF)],
          gsems=[pltpu.SemaphoreType.DMA for _ in range(_NBUF)],
          wsems=[pltpu.SemaphoreType.DMA for _ in range(_NBUF)],
      ),
      compiler_params=pltpu.CompilerParams(use_tc_tiling_on_sc=False),
  )
  return kern(table, x_flat)


def kernel(x, table):
  batch, hist = x.shape
  out = _embed(x.reshape(-1).astype(jnp.int32), table)
  return out.reshape(batch, hist, _EMBED_DIM)
